# baseline (speedup 1.0000x reference)
import functools

import jax
import jax.numpy as jnp
from jax import lax
from jax.experimental import pallas as pl
from jax.experimental.pallas import tpu as pltpu
from jax.experimental.pallas import tpu_sc as plsc

DIM = 32
B_ROWS = 16384
B_COLS = 26
VOCAB = 1000000
NW = 32
Q = 251904
TBR = 512
NTB = Q // TBR
ITILES = B_ROWS // 128
IT_PER_W = ITILES // NW
NITEMS = B_COLS * IT_PER_W

_mesh = plsc.VectorSubcoreMesh(core_axis_name="c", subcore_axis_name="s")


def _table_rm_body(p0, p1, p2, p3, out_ref):
    eye = (lax.broadcasted_iota(jnp.int32, (DIM, DIM), 0)
           == lax.broadcasted_iota(jnp.int32, (DIM, DIM), 1)
           ).astype(jnp.float32)
    dn = (((0,), (0,)), ((), ()))
    out_ref[...] = jnp.concatenate(
        [lax.dot_general(p[...], eye, dn,
                         precision=lax.Precision.HIGHEST,
                         preferred_element_type=jnp.float32)
         for p in (p0, p1, p2, p3)], axis=1)


def _table_rm(tbl_t):
    return pl.pallas_call(
        _table_rm_body,
        grid=(NTB,),
        in_specs=[
            pl.BlockSpec(
                (32, TBR),
                lambda j, m=m: (0, jnp.minimum(m * NTB + j, VOCAB // TBR)))
            for m in range(4)
        ],
        out_specs=pl.BlockSpec((TBR, 128), lambda j: (j, 0)),
        out_shape=jax.ShapeDtypeStruct((Q, 128), jnp.float32),
    )(tbl_t, tbl_t, tbl_t, tbl_t)


@functools.partial(
    pl.kernel,
    mesh=_mesh,
    compiler_params=pltpu.CompilerParams(
        use_tc_tiling_on_sc=False, needs_layout_passes=False),
    out_type=jax.ShapeDtypeStruct((B_COLS, 4, ITILES, 8, 128), jnp.float32),
    scratch_types=(
        [pltpu.VMEM((B_COLS, 8, 128), jnp.int32)]
        + [pltpu.VMEM((128, DIM), jnp.float32) for _ in range(2)]
        + [pltpu.VMEM((DIM, 128), jnp.float32) for _ in range(2)]
        + [pltpu.SemaphoreType.DMA for _ in range(4)]
    ),
)
def _gather_kernel(idx_hbm, scr_hbm, out_hbm,
                   idxv, rw0, rw1, ob0, ob1, gsm0, gsm1, osm0, osm1):
    wid = lax.axis_index("s") * 2 + lax.axis_index("c")
    iota = lax.iota(jnp.int32, 16)
    rws, obs = (rw0, rw1), (ob0, ob1)
    gsms, osms = (gsm0, gsm1), (osm0, osm1)

    pltpu.sync_copy(idx_hbm.at[:, pl.ds(8 * (wid // 2), 8)], idxv)
    rowbase = (wid % 2) * 4

    def item_jil(c):
        return c // IT_PER_W, lax.rem(c, IT_PER_W)

    def fire_gather(c, s):
        j, il = item_jil(c)
        pltpu.async_copy(
            scr_hbm.at[idxv.at[j, rowbase + il]], rws[s], gsms[s])

    def wait_gather(s):
        pltpu.make_async_copy(
            scr_hbm.at[idxv.at[0, 0]], rws[s], gsms[s]).wait()

    def transpose_item(s):
        for h in range(2):
            for r in range(16):
                dvec = 16 * h + lax.bitwise_and(iota + r, 15)
                for g in range(8):
                    lvec = iota + 16 * g
                    v = plsc.load_gather(rws[s], [lvec, dvec])
                    plsc.store_scatter(obs[s], [dvec, lvec], v)

    def start_out(c, s):
        j, il = item_jil(c)
        it = wid * IT_PER_W + il
        for b in range(4):
            pltpu.async_copy(
                obs[s].at[pl.ds(8 * b, 8)], out_hbm.at[j, b, it], osms[s])

    def wait_out(s):
        for _ in range(4):
            pltpu.make_async_copy(
                obs[s].at[pl.ds(0, 8)], out_hbm.at[0, 0, 0], osms[s]).wait()

    fire_gather(0, 0)

    def body(t, carry):
        for s in range(2):
            c = 2 * t + s

            @pl.when(c + 1 < NITEMS)
            def _():
                fire_gather(c + 1, 1 - s)
            wait_gather(s)

            @pl.when(c >= 2)
            def _():
                wait_out(s)
            transpose_item(s)
            start_out(c, s)
        return carry

    lax.fori_loop(0, NITEMS // 2, body, 0)
    wait_out(0)
    wait_out(1)


def kernel(inputs, table):
    tbl_t = jnp.transpose(table)
    idx = inputs.astype(jnp.int32)
    srow = 4 * lax.rem(idx, Q) + idx // Q
    idx3 = jnp.transpose(srow).reshape(B_COLS, ITILES, 128)
    scratch = _table_rm(tbl_t).reshape(4 * Q, DIM)
    out5 = _gather_kernel(idx3, scratch)
    return jnp.transpose(out5, (2, 4, 0, 1, 3)).reshape(
        B_ROWS, B_COLS, DIM)


# speedup vs baseline: 2.1497x; 2.1497x over previous
import functools

import jax
import jax.numpy as jnp
from jax import lax
from jax.experimental import pallas as pl
from jax.experimental.pallas import tpu as pltpu
from jax.experimental.pallas import tpu_sc as plsc

DIM = 32
B_ROWS = 16384
B_COLS = 26
VOCAB = 1000000
NW = 32
Q = 251904
TBR = 2048
NTB = Q // TBR
ITILES = B_ROWS // 128
IT_PER_W = ITILES // NW
NITEMS = B_COLS * IT_PER_W

_mesh = plsc.VectorSubcoreMesh(core_axis_name="c", subcore_axis_name="s")


def _table_rm_body(p0, p1, p2, p3, out_ref):
    eye = (lax.broadcasted_iota(jnp.int32, (128, 128), 0)
           == lax.broadcasted_iota(jnp.int32, (128, 128), 1)
           ).astype(jnp.float32)
    x = jnp.concatenate([p0[...], p1[...], p2[...], p3[...]], axis=0)
    out_ref[...] = lax.dot_general(
        x, eye, (((0,), (0,)), ((), ())),
        precision=lax.Precision.HIGHEST,
        preferred_element_type=jnp.float32)


def _table_rm(tbl_t):
    return pl.pallas_call(
        _table_rm_body,
        grid=(NTB,),
        in_specs=[
            pl.BlockSpec(
                (32, TBR),
                lambda j, m=m: (0, jnp.minimum(m * NTB + j, VOCAB // TBR)))
            for m in range(4)
        ],
        out_specs=pl.BlockSpec((TBR, 128), lambda j: (j, 0)),
        out_shape=jax.ShapeDtypeStruct((Q, 128), jnp.float32),
    )(tbl_t, tbl_t, tbl_t, tbl_t)


@functools.partial(
    pl.kernel,
    mesh=_mesh,
    compiler_params=pltpu.CompilerParams(
        use_tc_tiling_on_sc=False, needs_layout_passes=False),
    out_type=jax.ShapeDtypeStruct((B_COLS, 4, ITILES, 8, 128), jnp.float32),
    scratch_types=(
        [pltpu.VMEM((B_COLS, 8, 128), jnp.int32)]
        + [pltpu.VMEM((128, DIM), jnp.float32) for _ in range(2)]
        + [pltpu.VMEM((DIM, 128), jnp.float32) for _ in range(2)]
        + [pltpu.SemaphoreType.DMA for _ in range(4)]
    ),
)
def _gather_kernel(idx_hbm, scr_hbm, out_hbm,
                   idxv, rw0, rw1, ob0, ob1, gsm0, gsm1, osm0, osm1):
    wid = lax.axis_index("s") * 2 + lax.axis_index("c")
    iota = lax.iota(jnp.int32, 16)
    rws, obs = (rw0, rw1), (ob0, ob1)
    gsms, osms = (gsm0, gsm1), (osm0, osm1)

    pltpu.sync_copy(idx_hbm.at[:, pl.ds(8 * (wid // 2), 8)], idxv)
    rowbase = (wid % 2) * 4

    def item_jil(c):
        return c // IT_PER_W, lax.rem(c, IT_PER_W)

    def fire_gather(c, s):
        j, il = item_jil(c)
        pltpu.async_copy(
            scr_hbm.at[idxv.at[j, rowbase + il]], rws[s], gsms[s])

    def wait_gather(s):
        pltpu.make_async_copy(
            scr_hbm.at[idxv.at[0, 0]], rws[s], gsms[s]).wait()

    def transpose_item(s):
        for h in range(2):
            for r in range(16):
                dvec = 16 * h + lax.bitwise_and(iota + r, 15)
                for g in range(8):
                    lvec = iota + 16 * g
                    v = plsc.load_gather(rws[s], [lvec, dvec])
                    plsc.store_scatter(obs[s], [dvec, lvec], v)

    def start_out(c, s):
        j, il = item_jil(c)
        it = wid * IT_PER_W + il
        for b in range(4):
            pltpu.async_copy(
                obs[s].at[pl.ds(8 * b, 8)], out_hbm.at[j, b, it], osms[s])

    def wait_out(s):
        for _ in range(4):
            pltpu.make_async_copy(
                obs[s].at[pl.ds(0, 8)], out_hbm.at[0, 0, 0], osms[s]).wait()

    fire_gather(0, 0)

    def body(t, carry):
        for s in range(2):
            c = 2 * t + s

            @pl.when(c + 1 < NITEMS)
            def _():
                fire_gather(c + 1, 1 - s)
            wait_gather(s)

            @pl.when(c >= 2)
            def _():
                wait_out(s)
            transpose_item(s)
            start_out(c, s)
        return carry

    lax.fori_loop(0, NITEMS // 2, body, 0)
    wait_out(0)
    wait_out(1)


def kernel(inputs, table):
    tbl_t = jnp.transpose(table)
    idx = inputs.astype(jnp.int32)
    srow = 4 * lax.rem(idx, Q) + idx // Q
    idx3 = jnp.transpose(srow).reshape(B_COLS, ITILES, 128)
    scratch = _table_rm(tbl_t).reshape(4 * Q, DIM)
    out5 = _gather_kernel(idx3, scratch)
    return jnp.transpose(out5, (2, 4, 0, 1, 3)).reshape(
        B_ROWS, B_COLS, DIM)
